# baseline (speedup 1.0000x reference)
import functools

import jax
import jax.numpy as jnp
from jax import lax
from jax.experimental import pallas as pl
from jax.experimental.pallas import tpu as pltpu
from jax.experimental.pallas import tpu_sc as plsc

_G = 256


_CH = 128
_NBUF = 2
_NSPLIT = 2


def _sc_gather_scatter_add(table, gidx, sidx, n_rows, core_chunks):
    width = table.shape[1]
    rows_per_tile = n_rows // 16
    zblocks = rows_per_tile // _CH
    mesh = plsc.VectorSubcoreMesh(core_axis_name="c", subcore_axis_name="s")

    nbuf = _NBUF
    c0, c1 = core_chunks

    @functools.partial(
        pl.kernel,
        out_type=jax.ShapeDtypeStruct((2, n_rows, width), jnp.float32),
        mesh=mesh,
        scratch_types=(
            [pltpu.VMEM_SHARED((n_rows, width), jnp.float32),
             pltpu.VMEM((2, nbuf, _CH), jnp.int32),
             pltpu.VMEM((2, nbuf, _CH), jnp.int32)]
            + [pltpu.VMEM((_CH, width), jnp.float32) for _ in range(nbuf)]
            + [pltpu.SemaphoreType.DMA for _ in range(2 * nbuf + 2)]
        ),
    )
    def k(table_hbm, gidx_hbm, sidx_hbm, zero_hbm, out_hbm,
          acc_sh, gi_r, si_r, *bufs_sems):
        bufs = bufs_sems[:nbuf]
        gsems = bufs_sems[nbuf:2 * nbuf]
        ssems = bufs_sems[2 * nbuf:3 * nbuf]
        gisem = bufs_sems[3 * nbuf]
        sisem = bufs_sems[3 * nbuf + 1]
        cid = lax.axis_index("c")
        sid = lax.axis_index("s")
        base = sid * rows_per_tile
        n_chunks_c = jnp.where(cid == 0, c0, c1)
        n_rounds_c = n_chunks_c // nbuf
        for p in range(2):
            pltpu.sync_copy(gidx_hbm.at[cid, sid, pl.ds(p * nbuf, nbuf)],
                            gi_r.at[p])
            pltpu.sync_copy(sidx_hbm.at[cid, sid, pl.ds(p * nbuf, nbuf)],
                            si_r.at[p])
        pltpu.sync_copy(zero_hbm, bufs[0])
        for z in range(zblocks):
            pltpu.sync_copy(bufs[0], acc_sh.at[pl.ds(base + z * _CH, _CH)])
        plsc.subcore_barrier()

        hc = _CH // _NSPLIT

        def gather_chunk(p, b):
            for s in range(_NSPLIT):
                pltpu.async_copy(
                    table_hbm.at[gi_r.at[p, b, pl.ds(s * hc, hc)]],
                    bufs[b].at[pl.ds(s * hc, hc)], gsems[b])

        for b in range(nbuf):
            gather_chunk(0, b)

        def body(jj, carry):
            p = jj % 2
            for b in range(nbuf):
                pltpu.make_async_copy(
                    table_hbm.at[gi_r.at[0, 0]], bufs[b], gsems[b]).wait()
                pltpu.async_copy(bufs[b], acc_sh.at[si_r.at[p, b]],
                                 ssems[b], add=True)
            for b in range(nbuf):
                pltpu.make_async_copy(
                    bufs[b], acc_sh.at[si_r.at[0, 0]], ssems[b]).wait()

            @pl.when(jj >= 1)
            def _():
                pltpu.make_async_copy(
                    gidx_hbm.at[cid, sid, pl.ds(0, nbuf)], gi_r.at[0],
                    gisem).wait()
                pltpu.make_async_copy(
                    sidx_hbm.at[cid, sid, pl.ds(0, nbuf)], si_r.at[0],
                    sisem).wait()
            for b in range(nbuf):
                gather_chunk(1 - p, b)
            rs = jnp.minimum((jj + 2) * nbuf, n_chunks_c - nbuf)
            pltpu.async_copy(gidx_hbm.at[cid, sid, pl.ds(rs, nbuf)],
                             gi_r.at[p], gisem)
            pltpu.async_copy(sidx_hbm.at[cid, sid, pl.ds(rs, nbuf)],
                             si_r.at[p], sisem)
            return carry

        lax.fori_loop(0, n_rounds_c, body, 0)
        for b in range(nbuf):
            pltpu.make_async_copy(
                table_hbm.at[gi_r.at[0, 0]], bufs[b], gsems[b]).wait()
        pltpu.make_async_copy(
            gidx_hbm.at[cid, sid, pl.ds(0, nbuf)], gi_r.at[0], gisem).wait()
        pltpu.make_async_copy(
            sidx_hbm.at[cid, sid, pl.ds(0, nbuf)], si_r.at[0], sisem).wait()
        plsc.subcore_barrier()
        for z in range(zblocks):
            pltpu.sync_copy(acc_sh.at[pl.ds(base + z * _CH, _CH)], bufs[0])
            pltpu.sync_copy(bufs[0], out_hbm.at[cid, pl.ds(base + z * _CH, _CH)])

    zero = jnp.zeros((_CH, width), jnp.float32)
    return k(table, gidx, sidx, zero)


def _h0_kernel(x0r, x1r, table, n, bn):
    grid = n // bn

    def body(x0_ref, x1_ref, tab_ref, out_ref):
        x0 = x0_ref[0]
        x1 = x1_ref[0]
        it = lax.broadcasted_iota(jnp.int32, (128, bn), 0)
        ohT = jnp.logical_or(jnp.broadcast_to(x0, (128, bn)) == it,
                             jnp.broadcast_to(x1, (128, bn)) == it)
        oh = ohT.astype(jnp.float32)
        out_ref[...] = lax.dot_general(
            oh, tab_ref[...], (((0,), (0,)), ((), ())),
            preferred_element_type=jnp.float32)

    return pl.pallas_call(
        body,
        grid=(grid,),
        in_specs=[
            pl.BlockSpec((1, 1, bn), lambda i: (i, 0, 0)),
            pl.BlockSpec((1, 1, bn), lambda i: (i, 0, 0)),
            pl.BlockSpec((128, 128), lambda i: (0, 0)),
        ],
        out_specs=pl.BlockSpec((bn, 128), lambda i: (i, 0)),
        out_shape=jax.ShapeDtypeStruct((n, 128), jnp.float32),
    )(x0r, x1r, table)


def _mlp_kernel(parts, h, cparts, M, W1l, b1l, W2l, b2l, relu_out, n, bn):
    grid = n // bn

    def body(p_ref, h_ref, cp_ref, m_ref, w1_ref, b1_ref, w2_ref, b2_ref,
             out_ref):
        T = cp_ref[0] + cp_ref[1]
        agg = (p_ref[0] + p_ref[1] + h_ref[...]
               + jnp.dot(T, m_ref[...], preferred_element_type=jnp.float32)
               + m_ref[12:13, :])
        z = jnp.maximum(
            jnp.dot(agg, w1_ref[...], preferred_element_type=jnp.float32)
            + b1_ref[...], 0.0)
        o = jnp.dot(z, w2_ref[...], preferred_element_type=jnp.float32) \
            + b2_ref[...]
        if relu_out:
            o = jnp.maximum(o, 0.0)
        out_ref[...] = o

    return pl.pallas_call(
        body,
        grid=(grid,),
        in_specs=[
            pl.BlockSpec((2, bn, 128), lambda i: (0, i, 0)),
            pl.BlockSpec((bn, 128), lambda i: (i, 0)),
            pl.BlockSpec((2, bn, 128), lambda i: (0, i, 0)),
            pl.BlockSpec((128, 128), lambda i: (0, 0)),
            pl.BlockSpec((128, 256), lambda i: (0, 0)),
            pl.BlockSpec((1, 256), lambda i: (0, 0)),
            pl.BlockSpec((256, 128), lambda i: (0, 0)),
            pl.BlockSpec((1, 128), lambda i: (0, 0)),
        ],
        out_specs=pl.BlockSpec((bn, 128), lambda i: (i, 0)),
        out_shape=jax.ShapeDtypeStruct((n, 128), jnp.float32),
    )(parts, h, cparts, M, W1l, b1l, W2l, b2l)


def _pool_head_kernel(batchr, h, feat_W, feat_b, pW1, pb1, pW2, pb2, pW3, pb3,
                      n, bn):
    grid = n // bn

    def body(b_ref, h_ref, fw_ref, fb_ref, w1_ref, b1_ref, w2_ref, b2_ref,
             w3_ref, b3_ref, hf_ref, proj_ref, pooled):
        i = pl.program_id(0)
        b = b_ref[0]
        it = lax.broadcasted_iota(jnp.int32, (_G, bn), 0)
        ohT = (jnp.broadcast_to(b, (_G, bn)) == it).astype(jnp.float32)
        part = jnp.dot(ohT, h_ref[...], preferred_element_type=jnp.float32)

        @pl.when(i == 0)
        def _():
            pooled[...] = part

        @pl.when(i > 0)
        def _():
            pooled[...] = pooled[...] + part

        @pl.when(i == grid - 1)
        def _():
            hf = jnp.dot(pooled[...], fw_ref[...],
                         preferred_element_type=jnp.float32) + fb_ref[...]
            hf_ref[...] = hf
            p = jnp.maximum(
                jnp.dot(hf, w1_ref[...], preferred_element_type=jnp.float32)
                + b1_ref[...], 0.0)
            p = jnp.maximum(
                jnp.dot(p, w2_ref[...], preferred_element_type=jnp.float32)
                + b2_ref[...], 0.0)
            proj_ref[...] = jnp.dot(
                p, w3_ref[...], preferred_element_type=jnp.float32) \
                + b3_ref[...]

    return pl.pallas_call(
        body,
        grid=(grid,),
        in_specs=[
            pl.BlockSpec((1, 1, bn), lambda i: (i, 0, 0)),
            pl.BlockSpec((bn, 128), lambda i: (i, 0)),
            pl.BlockSpec((128, 256), lambda i: (0, 0)),
            pl.BlockSpec((1, 256), lambda i: (0, 0)),
            pl.BlockSpec((256, 128), lambda i: (0, 0)),
            pl.BlockSpec((1, 128), lambda i: (0, 0)),
            pl.BlockSpec((128, 128), lambda i: (0, 0)),
            pl.BlockSpec((1, 128), lambda i: (0, 0)),
            pl.BlockSpec((128, 128), lambda i: (0, 0)),
            pl.BlockSpec((1, 128), lambda i: (0, 0)),
        ],
        out_specs=[
            pl.BlockSpec((_G, 256), lambda i: (0, 0)),
            pl.BlockSpec((_G, 128), lambda i: (0, 0)),
        ],
        out_shape=[
            jax.ShapeDtypeStruct((_G, 256), jnp.float32),
            jax.ShapeDtypeStruct((_G, 128), jnp.float32),
        ],
        scratch_shapes=[pltpu.VMEM((_G, 128), jnp.float32)],
    )(batchr, h, feat_W, feat_b, pW1, pb1, pW2, pb2, pW3, pb3)


def kernel(x, edge_index, edge_attr, batch, x_emb1, x_emb2, edge_emb1,
           edge_emb2, W1, b1, W2, b2, feat_W, feat_b, pW1, pb1, pW2, pb2,
           pW3, pb3):
    n = x.shape[0]
    e = edge_index.shape[1]
    num_layer = W1.shape[0]
    bn = 1000

    unit = 16 * _CH
    c1 = _NBUF * max(2, round(0.25 * e / unit / _NBUF))
    c0 = _NBUF * (-(-(e - unit * c1) // (unit * _NBUF)))
    ep = unit * (c0 + c1)
    pad = ep - e
    i32 = jnp.int32
    n_rows = 2048 * (-(-(n + 1) // 2048))
    src = jnp.concatenate([edge_index[0], jnp.zeros((pad,), i32)])
    pad_dst = n + jnp.arange(pad, dtype=i32) % (n_rows - n)
    dst = jnp.concatenate([edge_index[1], pad_dst])
    t = edge_attr[:, 0] * 3 + edge_attr[:, 1]
    tp = jnp.concatenate([t, jnp.zeros((pad,), i32)])
    reps = 640
    tp = tp + 16 * (jnp.arange(ep, dtype=i32) % reps)

    def _slab(a, k0, k1):
        cm = max(k0, k1)
        a0 = jnp.pad(a[:unit * k0].reshape(16, k0, _CH),
                     ((0, 0), (0, cm - k0), (0, 0)))
        a1 = jnp.pad(a[unit * k0:].reshape(16, k1, _CH),
                     ((0, 0), (0, cm - k1), (0, 0)))
        return jnp.stack([a0, a1])

    gidx_h = _slab(src, c0, c1)
    sidx = _slab(dst, c0, c1)
    core_chunks = (c0, c1)
    ch = _NBUF * ((c0 + c1) // 2 // _NBUF)
    cc0, cc1 = c0 + c1 - ch, ch
    gidx_tc = _slab(tp, cc0, cc1)
    sidx_c = _slab(dst, cc0, cc1)
    count_chunks = (cc0, cc1)

    na = x_emb1.shape[0]
    d = x_emb1.shape[1]
    table = jnp.zeros((128, d), jnp.float32)
    table = lax.dynamic_update_slice(table, x_emb1, (0, 0))
    table = lax.dynamic_update_slice(table, x_emb2, (na, 0))
    x0r = x[:, 0].reshape(-1, 1, bn)
    x1r = (x[:, 1] + na).reshape(-1, 1, bn)

    eye16 = jnp.tile(jnp.eye(16, 128, dtype=jnp.float32), (reps, 1))
    cparts = _sc_gather_scatter_add(eye16, gidx_tc, sidx_c, n_rows,
                                    count_chunks)

    h = _h0_kernel(x0r, x1r, table, n, bn)

    for l in range(num_layer):
        Ml = (jnp.repeat(edge_emb1[l], 3, axis=0)[:15]
              + jnp.tile(edge_emb2[l], (5, 1))[:15])
        Ml = jnp.concatenate([Ml, jnp.zeros((113, d), jnp.float32)], axis=0)
        parts = _sc_gather_scatter_add(h, gidx_h, sidx, n_rows, core_chunks)
        h = _mlp_kernel(parts, h, cparts, Ml, W1[l], b1[l][None, :], W2[l],
                        b2[l][None, :], l < num_layer - 1, n, bn)

    batchr = batch.reshape(-1, 1, bn)
    hf, proj = _pool_head_kernel(batchr, h, feat_W, feat_b[None, :],
                                 pW1, pb1[None, :], pW2, pb2[None, :],
                                 pW3, pb3[None, :], n, bn)
    return (hf, proj)


# speedup vs baseline: 1.0081x; 1.0081x over previous
import functools

import jax
import jax.numpy as jnp
from jax import lax
from jax.experimental import pallas as pl
from jax.experimental.pallas import tpu as pltpu
from jax.experimental.pallas import tpu_sc as plsc

_G = 256


_CH = 128
_NBUF = 2
_NSPLIT = 2


def _sc_gather_scatter_add(table, gidx, sidx, n_rows, core_chunks):
    width = table.shape[1]
    rows_per_tile = n_rows // 16
    zblocks = rows_per_tile // _CH
    mesh = plsc.VectorSubcoreMesh(core_axis_name="c", subcore_axis_name="s")

    nbuf = _NBUF
    c0, c1 = core_chunks

    @functools.partial(
        pl.kernel,
        out_type=jax.ShapeDtypeStruct((2, n_rows, width), jnp.float32),
        mesh=mesh,
        scratch_types=(
            [pltpu.VMEM_SHARED((n_rows, width), jnp.float32),
             pltpu.VMEM((2, nbuf, _CH), jnp.int32),
             pltpu.VMEM((2, nbuf, _CH), jnp.int32)]
            + [pltpu.VMEM((_CH, width), jnp.float32) for _ in range(nbuf)]
            + [pltpu.SemaphoreType.DMA for _ in range(2 * nbuf + 2)]
        ),
    )
    def k(table_hbm, gidx_hbm, sidx_hbm, zero_hbm, out_hbm,
          acc_sh, gi_r, si_r, *bufs_sems):
        bufs = bufs_sems[:nbuf]
        gsems = bufs_sems[nbuf:2 * nbuf]
        ssems = bufs_sems[2 * nbuf:3 * nbuf]
        gisem = bufs_sems[3 * nbuf]
        sisem = bufs_sems[3 * nbuf + 1]
        cid = lax.axis_index("c")
        sid = lax.axis_index("s")
        base = sid * rows_per_tile
        n_chunks_c = jnp.where(cid == 0, c0, c1)
        n_rounds_c = n_chunks_c // nbuf
        for p in range(2):
            pltpu.sync_copy(gidx_hbm.at[cid, sid, pl.ds(p * nbuf, nbuf)],
                            gi_r.at[p])
            pltpu.sync_copy(sidx_hbm.at[cid, sid, pl.ds(p * nbuf, nbuf)],
                            si_r.at[p])
        pltpu.sync_copy(zero_hbm, bufs[0])
        for z in range(zblocks):
            pltpu.sync_copy(bufs[0], acc_sh.at[pl.ds(base + z * _CH, _CH)])
        plsc.subcore_barrier()

        hc = _CH // _NSPLIT

        def gather_chunk(p, b):
            for s in range(_NSPLIT):
                pltpu.async_copy(
                    table_hbm.at[gi_r.at[p, b, pl.ds(s * hc, hc)]],
                    bufs[b].at[pl.ds(s * hc, hc)], gsems[b])

        for b in range(nbuf):
            gather_chunk(0, b)

        def body(jj, carry):
            p = jj % 2
            for b in range(nbuf):
                pltpu.make_async_copy(
                    table_hbm.at[gi_r.at[0, 0]], bufs[b], gsems[b]).wait()
                pltpu.async_copy(bufs[b], acc_sh.at[si_r.at[p, b]],
                                 ssems[b], add=True)
            for b in range(nbuf):
                pltpu.make_async_copy(
                    bufs[b], acc_sh.at[si_r.at[0, 0]], ssems[b]).wait()

            @pl.when(jj >= 1)
            def _():
                pltpu.make_async_copy(
                    gidx_hbm.at[cid, sid, pl.ds(0, nbuf)], gi_r.at[0],
                    gisem).wait()
                pltpu.make_async_copy(
                    sidx_hbm.at[cid, sid, pl.ds(0, nbuf)], si_r.at[0],
                    sisem).wait()
            for b in range(nbuf):
                gather_chunk(1 - p, b)
            rs = jnp.minimum((jj + 2) * nbuf, n_chunks_c - nbuf)
            pltpu.async_copy(gidx_hbm.at[cid, sid, pl.ds(rs, nbuf)],
                             gi_r.at[p], gisem)
            pltpu.async_copy(sidx_hbm.at[cid, sid, pl.ds(rs, nbuf)],
                             si_r.at[p], sisem)
            return carry

        lax.fori_loop(0, n_rounds_c, body, 0)
        for b in range(nbuf):
            pltpu.make_async_copy(
                table_hbm.at[gi_r.at[0, 0]], bufs[b], gsems[b]).wait()
        pltpu.make_async_copy(
            gidx_hbm.at[cid, sid, pl.ds(0, nbuf)], gi_r.at[0], gisem).wait()
        pltpu.make_async_copy(
            sidx_hbm.at[cid, sid, pl.ds(0, nbuf)], si_r.at[0], sisem).wait()
        plsc.subcore_barrier()
        for z in range(zblocks):
            pltpu.sync_copy(acc_sh.at[pl.ds(base + z * _CH, _CH)], bufs[0])
            pltpu.sync_copy(bufs[0], out_hbm.at[cid, pl.ds(base + z * _CH, _CH)])

    zero = jnp.zeros((_CH, width), jnp.float32)
    return k(table, gidx, sidx, zero)


def _h0_kernel(x0r, x1r, table, n, bn):
    grid = n // bn

    def body(x0_ref, x1_ref, tab_ref, out_ref):
        x0 = x0_ref[0]
        x1 = x1_ref[0]
        it = lax.broadcasted_iota(jnp.int32, (128, bn), 0)
        ohT = jnp.logical_or(jnp.broadcast_to(x0, (128, bn)) == it,
                             jnp.broadcast_to(x1, (128, bn)) == it)
        oh = ohT.astype(jnp.float32)
        out_ref[...] = lax.dot_general(
            oh, tab_ref[...], (((0,), (0,)), ((), ())),
            preferred_element_type=jnp.float32)

    return pl.pallas_call(
        body,
        grid=(grid,),
        in_specs=[
            pl.BlockSpec((1, 1, bn), lambda i: (i, 0, 0)),
            pl.BlockSpec((1, 1, bn), lambda i: (i, 0, 0)),
            pl.BlockSpec((128, 128), lambda i: (0, 0)),
        ],
        out_specs=pl.BlockSpec((bn, 128), lambda i: (i, 0)),
        out_shape=jax.ShapeDtypeStruct((n, 128), jnp.float32),
    )(x0r, x1r, table)


def _mlp_kernel(parts, h, cparts, M, W1l, b1l, W2l, b2l, relu_out, n, bn):
    grid = n // bn

    def body(p_ref, h_ref, cp_ref, m_ref, w1_ref, b1_ref, w2_ref, b2_ref,
             out_ref):
        T = cp_ref[0] + cp_ref[1]
        agg = (p_ref[0] + p_ref[1] + h_ref[...]
               + jnp.dot(T, m_ref[...], preferred_element_type=jnp.float32)
               + m_ref[12:13, :])
        z = jnp.maximum(
            jnp.dot(agg, w1_ref[...], preferred_element_type=jnp.float32)
            + b1_ref[...], 0.0)
        o = jnp.dot(z, w2_ref[...], preferred_element_type=jnp.float32) \
            + b2_ref[...]
        if relu_out:
            o = jnp.maximum(o, 0.0)
        out_ref[...] = o

    return pl.pallas_call(
        body,
        grid=(grid,),
        in_specs=[
            pl.BlockSpec((2, bn, 128), lambda i: (0, i, 0)),
            pl.BlockSpec((bn, 128), lambda i: (i, 0)),
            pl.BlockSpec((2, bn, 128), lambda i: (0, i, 0)),
            pl.BlockSpec((128, 128), lambda i: (0, 0)),
            pl.BlockSpec((128, 256), lambda i: (0, 0)),
            pl.BlockSpec((1, 256), lambda i: (0, 0)),
            pl.BlockSpec((256, 128), lambda i: (0, 0)),
            pl.BlockSpec((1, 128), lambda i: (0, 0)),
        ],
        out_specs=pl.BlockSpec((bn, 128), lambda i: (i, 0)),
        out_shape=jax.ShapeDtypeStruct((n, 128), jnp.float32),
    )(parts, h, cparts, M, W1l, b1l, W2l, b2l)


def _pool_head_kernel(batchr, h, feat_W, feat_b, pW1, pb1, pW2, pb2, pW3, pb3,
                      n, bn):
    grid = n // bn

    def body(b_ref, h_ref, fw_ref, fb_ref, w1_ref, b1_ref, w2_ref, b2_ref,
             w3_ref, b3_ref, hf_ref, proj_ref, pooled):
        i = pl.program_id(0)
        b = b_ref[0]
        it = lax.broadcasted_iota(jnp.int32, (_G, bn), 0)
        ohT = (jnp.broadcast_to(b, (_G, bn)) == it).astype(jnp.float32)
        part = jnp.dot(ohT, h_ref[...], preferred_element_type=jnp.float32)

        @pl.when(i == 0)
        def _():
            pooled[...] = part

        @pl.when(i > 0)
        def _():
            pooled[...] = pooled[...] + part

        @pl.when(i == grid - 1)
        def _():
            hf = jnp.dot(pooled[...], fw_ref[...],
                         preferred_element_type=jnp.float32) + fb_ref[...]
            hf_ref[...] = hf
            p = jnp.maximum(
                jnp.dot(hf, w1_ref[...], preferred_element_type=jnp.float32)
                + b1_ref[...], 0.0)
            p = jnp.maximum(
                jnp.dot(p, w2_ref[...], preferred_element_type=jnp.float32)
                + b2_ref[...], 0.0)
            proj_ref[...] = jnp.dot(
                p, w3_ref[...], preferred_element_type=jnp.float32) \
                + b3_ref[...]

    return pl.pallas_call(
        body,
        grid=(grid,),
        in_specs=[
            pl.BlockSpec((1, 1, bn), lambda i: (i, 0, 0)),
            pl.BlockSpec((bn, 128), lambda i: (i, 0)),
            pl.BlockSpec((128, 256), lambda i: (0, 0)),
            pl.BlockSpec((1, 256), lambda i: (0, 0)),
            pl.BlockSpec((256, 128), lambda i: (0, 0)),
            pl.BlockSpec((1, 128), lambda i: (0, 0)),
            pl.BlockSpec((128, 128), lambda i: (0, 0)),
            pl.BlockSpec((1, 128), lambda i: (0, 0)),
            pl.BlockSpec((128, 128), lambda i: (0, 0)),
            pl.BlockSpec((1, 128), lambda i: (0, 0)),
        ],
        out_specs=[
            pl.BlockSpec((_G, 256), lambda i: (0, 0)),
            pl.BlockSpec((_G, 128), lambda i: (0, 0)),
        ],
        out_shape=[
            jax.ShapeDtypeStruct((_G, 256), jnp.float32),
            jax.ShapeDtypeStruct((_G, 128), jnp.float32),
        ],
        scratch_shapes=[pltpu.VMEM((_G, 128), jnp.float32)],
    )(batchr, h, feat_W, feat_b, pW1, pb1, pW2, pb2, pW3, pb3)


def kernel(x, edge_index, edge_attr, batch, x_emb1, x_emb2, edge_emb1,
           edge_emb2, W1, b1, W2, b2, feat_W, feat_b, pW1, pb1, pW2, pb2,
           pW3, pb3):
    n = x.shape[0]
    e = edge_index.shape[1]
    num_layer = W1.shape[0]
    bn = 1000

    unit = 16 * _CH
    c1 = _NBUF * max(2, round(0.25 * e / unit / _NBUF))
    c0 = _NBUF * (-(-(e - unit * c1) // (unit * _NBUF)))
    ep = unit * (c0 + c1)
    pad = ep - e
    i32 = jnp.int32
    n_rows = 2048 * (-(-(n + 1) // 2048))
    src = jnp.concatenate([edge_index[0], jnp.zeros((pad,), i32)])
    pad_dst = n + jnp.arange(pad, dtype=i32) % (n_rows - n)
    dst = jnp.concatenate([edge_index[1], pad_dst])
    t = edge_attr[:, 0] * 3 + edge_attr[:, 1]
    tp = jnp.concatenate([t, jnp.zeros((pad,), i32)])
    reps = 640
    tp = tp + 16 * (jnp.arange(ep, dtype=i32) % reps)

    def _slab(a, k0, k1):
        cm = max(k0, k1)
        a0 = jnp.pad(a[:unit * k0].reshape(16, k0, _CH),
                     ((0, 0), (0, cm - k0), (0, 0)))
        a1 = jnp.pad(a[unit * k0:].reshape(16, k1, _CH),
                     ((0, 0), (0, cm - k1), (0, 0)))
        return jnp.stack([a0, a1])

    gidx_h = _slab(src, c0, c1)
    sidx = _slab(dst, c0, c1)
    gidx_t = _slab(tp, c0, c1)
    core_chunks = (c0, c1)

    na = x_emb1.shape[0]
    d = x_emb1.shape[1]
    table = jnp.zeros((128, d), jnp.float32)
    table = lax.dynamic_update_slice(table, x_emb1, (0, 0))
    table = lax.dynamic_update_slice(table, x_emb2, (na, 0))
    x0r = x[:, 0].reshape(-1, 1, bn)
    x1r = (x[:, 1] + na).reshape(-1, 1, bn)

    eye16 = jnp.tile(jnp.eye(16, 128, dtype=jnp.float32), (reps, 1))
    cparts = _sc_gather_scatter_add(eye16, gidx_t, sidx, n_rows, core_chunks)

    h = _h0_kernel(x0r, x1r, table, n, bn)

    for l in range(num_layer):
        Ml = (jnp.repeat(edge_emb1[l], 3, axis=0)[:15]
              + jnp.tile(edge_emb2[l], (5, 1))[:15])
        Ml = jnp.concatenate([Ml, jnp.zeros((113, d), jnp.float32)], axis=0)
        parts = _sc_gather_scatter_add(h, gidx_h, sidx, n_rows, core_chunks)
        h = _mlp_kernel(parts, h, cparts, Ml, W1[l], b1[l][None, :], W2[l],
                        b2[l][None, :], l < num_layer - 1, n, bn)

    batchr = batch.reshape(-1, 1, bn)
    hf, proj = _pool_head_kernel(batchr, h, feat_W, feat_b[None, :],
                                 pW1, pb1[None, :], pW2, pb2[None, :],
                                 pW3, pb3[None, :], n, bn)
    return (hf, proj)
